# baseline (speedup 1.0000x reference)
import dataclasses
import functools

import jax
import jax.numpy as jnp
from jax import lax
from jax.experimental import pallas as pl
from jax.experimental.pallas import tpu as pltpu
from jax.experimental.pallas import tpu_sc as plsc

_B0, _B1 = 16384, 50
_D = 32
_W = 256
_NC = _B0 // _W
_PACK = 128 // _D


def _sc_compiler_params():
    cp = pltpu.CompilerParams()
    if "needs_layout_passes" in pltpu.CompilerParams.__dataclass_fields__:
        cp = dataclasses.replace(cp, needs_layout_passes=False)
    return cp


def kernel(indices, weight):
    w4 = weight.reshape(1000000 // _PACK, 128)
    i_t = indices.T

    mesh = plsc.VectorSubcoreMesh(
        core_axis_name="core", subcore_axis_name="subcore"
    )

    @functools.partial(
        pl.kernel,
        out_type=jax.ShapeDtypeStruct((_B1, _D, _B0), weight.dtype),
        mesh=mesh,
        scratch_types=[
            pltpu.VMEM((_W, 128), jnp.float32),
            pltpu.VMEM((_W,), jnp.int32),
            pltpu.SemaphoreType.DMA,
        ],
        compiler_params=_sc_compiler_params(),
    )
    def k(w_hbm, i_hbm, o_hbm, g_v, p_v, sem):
        def body(i_vmem, o_vmem):
            for c in range(_W // 16):
                v = i_vmem[0, pl.ds(c * 16, 16)]
                p_v[pl.ds(c * 16, 16)] = v >> 2
            pltpu.async_copy(w_hbm.at[p_v], g_v, sem).wait()
            @pl.loop(0, _D)
            def _(f):
                for c in range(_W // 16):
                    v = i_vmem[0, pl.ds(c * 16, 16)]
                    col = (v & 3) * _D + f
                    row = lax.iota(jnp.int32, 16) + c * 16
                    o_vmem[0, f, pl.ds(c * 16, 16)] = plsc.load_gather(
                        g_v, [row, col]
                    )

        pltpu.emit_pipeline(
            body,
            grid=(_B1 * _NC,),
            in_specs=[
                pl.BlockSpec((1, _W), index_map=lambda i: (i // _NC, i % _NC))
            ],
            out_specs=[
                pl.BlockSpec(
                    (1, _D, _W), index_map=lambda i: (i // _NC, 0, i % _NC)
                )
            ],
            core_axis_name=("core", "subcore"),
            dimension_semantics=(pltpu.PARALLEL,),
        )(i_hbm, o_hbm)

    out_t = k(w4, i_t)
    return out_t.transpose(2, 0, 1)


# speedup vs baseline: 1.3447x; 1.3447x over previous
import dataclasses
import functools

import jax
import jax.numpy as jnp
from jax import lax
from jax.experimental import pallas as pl
from jax.experimental.pallas import tpu as pltpu
from jax.experimental.pallas import tpu_sc as plsc

_B0, _B1 = 16384, 50
_D = 32
_W = 256
_NC = _B0 // _W
_PACK = 128 // _D


def _sc_compiler_params():
    cp = pltpu.CompilerParams()
    if "needs_layout_passes" in pltpu.CompilerParams.__dataclass_fields__:
        cp = dataclasses.replace(cp, needs_layout_passes=False)
    return cp


def kernel(indices, weight):
    w4 = weight.reshape(1000000 // _PACK, 128)
    i_t = indices.T

    mesh = plsc.VectorSubcoreMesh(
        core_axis_name="core", subcore_axis_name="subcore"
    )

    @functools.partial(
        pl.kernel,
        out_type=jax.ShapeDtypeStruct((_B1, _D, _B0), weight.dtype),
        mesh=mesh,
        scratch_types=[
            pltpu.VMEM((_W, 128), jnp.float32),
            pltpu.VMEM((_W,), jnp.int32),
            pltpu.SemaphoreType.DMA,
        ],
        compiler_params=_sc_compiler_params(),
    )
    def k(w_hbm, i_hbm, o_hbm, g_v, p_v, sem):
        def body(i_vmem, o_vmem):
            for c in range(_W // 16):
                v = i_vmem[0, pl.ds(c * 16, 16)]
                p_v[pl.ds(c * 16, 16)] = v >> 2
            pltpu.async_copy(w_hbm.at[p_v], g_v, sem).wait()
            for c in range(_W // 16):
                v = i_vmem[0, pl.ds(c * 16, 16)]
                cb = (v & 3) * _D
                row = lax.iota(jnp.int32, 16) + c * 16
                for f in range(_D):
                    o_vmem[0, f, pl.ds(c * 16, 16)] = plsc.load_gather(
                        g_v, [row, cb + f]
                    )

        pltpu.emit_pipeline(
            body,
            grid=(_B1 * _NC,),
            in_specs=[
                pl.BlockSpec((1, _W), index_map=lambda i: (i // _NC, i % _NC))
            ],
            out_specs=[
                pl.BlockSpec(
                    (1, _D, _W), index_map=lambda i: (i // _NC, 0, i % _NC)
                )
            ],
            core_axis_name=("core", "subcore"),
            dimension_semantics=(pltpu.PARALLEL,),
        )(i_hbm, o_hbm)

    out_t = k(w4, i_t)
    return out_t.transpose(2, 0, 1)


# speedup vs baseline: 1.6688x; 1.2410x over previous
import dataclasses
import functools

import jax
import jax.numpy as jnp
from jax import lax
from jax.experimental import pallas as pl
from jax.experimental.pallas import tpu as pltpu
from jax.experimental.pallas import tpu_sc as plsc

_B0, _B1 = 16384, 50
_D = 32
_W = 256
_NC = _B0 // _W
_PACK = 128 // _D


def _sc_compiler_params():
    cp = pltpu.CompilerParams()
    if "needs_layout_passes" in pltpu.CompilerParams.__dataclass_fields__:
        cp = dataclasses.replace(cp, needs_layout_passes=False)
    return cp


def kernel(indices, weight):
    w4 = weight.reshape(1000000 // _PACK, 128)
    i_t = indices.T

    mesh = plsc.VectorSubcoreMesh(
        core_axis_name="core", subcore_axis_name="subcore"
    )

    @functools.partial(
        pl.kernel,
        out_type=jax.ShapeDtypeStruct((_B1, _D, _B0), weight.dtype),
        mesh=mesh,
        scratch_types=[
            pltpu.VMEM((_W, 128), jnp.float32),
            pltpu.VMEM((_W,), jnp.int32),
            pltpu.SemaphoreType.DMA,
        ],
        compiler_params=_sc_compiler_params(),
    )
    def k(w_hbm, i_hbm, o_hbm, g_v, p_v, sem):
        def body(i_vmem, o_vmem):
            for c in range(_W // 16):
                v = i_vmem[0, pl.ds(c * 16, 16)]
                p_v[pl.ds(c * 16, 16)] = v >> 2
            pltpu.async_copy(w_hbm.at[p_v], g_v, sem).wait()
            @plsc.parallel_loop(0, _W // 16, unroll=2)
            def _(c):
                v = i_vmem[0, pl.ds(c * 16, 16)]
                cb = (v & 3) * _D
                row = lax.iota(jnp.int32, 16) + c * 16
                for f in range(_D):
                    o_vmem[0, f, pl.ds(c * 16, 16)] = plsc.load_gather(
                        g_v, [row, cb + f]
                    )

        pltpu.emit_pipeline(
            body,
            grid=(_B1 * _NC,),
            in_specs=[
                pl.BlockSpec((1, _W), index_map=lambda i: (i // _NC, i % _NC))
            ],
            out_specs=[
                pl.BlockSpec(
                    (1, _D, _W), index_map=lambda i: (i // _NC, 0, i % _NC)
                )
            ],
            core_axis_name=("core", "subcore"),
            dimension_semantics=(pltpu.PARALLEL,),
        )(i_hbm, o_hbm)

    out_t = k(w4, i_t)
    return out_t.transpose(2, 0, 1)
